# baseline (speedup 1.0000x reference)
import jax
import jax.numpy as jnp
from jax.experimental import pallas as pl
from jax.experimental.pallas import tpu as pltpu

_B = 1024
_S = 4096
_I = 256
_BJ = 512
_H = _BJ // 2
_NJ = _S // _BJ


def _esn_tile(inputs_ref, states_ref, win_ref, wres_ref, out_ref, sb_ref):
    t = pl.program_id(0)

    @pl.when(t == 0)
    def _cast_states():
        sb_ref[...] = states_ref[...].astype(jnp.bfloat16)

    sb = sb_ref[...]
    ib = inputs_ref[...].astype(jnp.bfloat16)
    for h in range(2):
        wb = wres_ref[:, pl.ds(h * _H, _H)].astype(jnp.bfloat16)
        winb = win_ref[:, pl.ds(h * _H, _H)].astype(jnp.bfloat16)
        z = jnp.dot(sb, wb, preferred_element_type=jnp.float32)
        z = z + jnp.dot(ib, winb, preferred_element_type=jnp.float32)
        cand = jnp.tanh(z)
        sj = states_ref[:, pl.ds(t * _BJ + h * _H, _H)]
        out_ref[:, pl.ds(h * _H, _H)] = sj + (cand - sj)


def kernel(inputs, states, Win, Wres):
    return pl.pallas_call(
        _esn_tile,
        grid=(_NJ,),
        in_specs=[
            pl.BlockSpec((_B, _I), lambda t: (0, 0)),
            pl.BlockSpec((_B, _S), lambda t: (0, 0)),
            pl.BlockSpec((_I, _BJ), lambda t: (0, t)),
            pl.BlockSpec((_S, _BJ), lambda t: (0, t)),
        ],
        out_specs=pl.BlockSpec((_B, _BJ), lambda t: (0, t)),
        out_shape=jax.ShapeDtypeStruct((_B, _S), jnp.float32),
        scratch_shapes=[pltpu.VMEM((_B, _S), jnp.bfloat16)],
    )(inputs, states, Win, Wres)
